# Initial kernel scaffold; baseline (speedup 1.0000x reference)
import jax
import jax.numpy as jnp
from jax.experimental import pallas as pl


def kernel(hidden_states, gate_w, w_gate_up, w_down, sh_gate_up, sh_down, sh_gate):
    raise NotImplementedError("write your pallas kernel here")


# speedup vs baseline: 1.4267x; 1.4267x over previous
import functools

import jax
import jax.numpy as jnp
from jax import lax
from jax.experimental import pallas as pl
from jax.experimental.pallas import tpu as pltpu
from jax.experimental.pallas import tpu_sc as plsc

_SC_WORKERS = 32


def _router_body(x_ref, gw_ref, w_ref, id_ref):
    x = x_ref[...]
    gw = gw_ref[...]
    logits = lax.dot_general(x, gw, (((1,), (1,)), ((), ())),
                             preferred_element_type=jnp.float32)
    m = jnp.max(logits, axis=1, keepdims=True)
    ex = jnp.exp(logits - m)
    p = ex / jnp.sum(ex, axis=1, keepdims=True)
    ncols = p.shape[1]
    iota = lax.broadcasted_iota(jnp.int32, p.shape, 1)
    p1 = jnp.max(p, axis=1, keepdims=True)
    i1 = jnp.min(jnp.where(p >= p1, iota, ncols), axis=1, keepdims=True)
    pm = jnp.where(iota == i1, -1.0, p)
    p2 = jnp.max(pm, axis=1, keepdims=True)
    i2 = jnp.min(jnp.where(pm >= p2, iota, ncols), axis=1, keepdims=True)
    s = p1 + p2
    w_ref[...] = jnp.concatenate([p1 / s, p2 / s], axis=1)
    id_ref[...] = jnp.concatenate([i1, i2], axis=1).astype(jnp.int32)


def _run_router(x, gate_w, br):
    t, d = x.shape
    e = gate_w.shape[0]
    return pl.pallas_call(
        _router_body,
        grid=(t // br,),
        in_specs=[
            pl.BlockSpec((br, d), lambda i: (i, 0)),
            pl.BlockSpec((e, d), lambda i: (0, 0)),
        ],
        out_specs=[
            pl.BlockSpec((br, 2), lambda i: (i, 0)),
            pl.BlockSpec((br, 2), lambda i: (i, 0)),
        ],
        out_shape=[
            jax.ShapeDtypeStruct((t, 2), jnp.float32),
            jax.ShapeDtypeStruct((t, 2), jnp.int32),
        ],
    )(x, gate_w)


def _sc_gather(table, idx, nrows, d, chunk):
    b_per_w = nrows // _SC_WORKERS
    n_ch = b_per_w // chunk
    mesh = plsc.VectorSubcoreMesh(core_axis_name="c", subcore_axis_name="s")

    @functools.partial(
        pl.kernel,
        mesh=mesh,
        out_type=jax.ShapeDtypeStruct((nrows, d), jnp.float32),
        scratch_types=[
            pltpu.VMEM((chunk,), jnp.int32),
            pltpu.VMEM((chunk, d), jnp.float32),
            pltpu.SemaphoreType.DMA,
        ],
    )
    def gk(table_hbm, idx_hbm, out_hbm, idx_v, rows_v, sem):
        wid = lax.axis_index("s") * 2 + lax.axis_index("c")
        base = wid * b_per_w
        for c in range(n_ch):
            off = base + c * chunk
            pltpu.sync_copy(idx_hbm.at[pl.ds(off, chunk)], idx_v)
            pltpu.async_copy(table_hbm.at[idx_v], rows_v, sem).wait()
            pltpu.sync_copy(rows_v, out_hbm.at[pl.ds(off, chunk)])

    return gk(table, idx)


def _moe_body(te_ref, w_ref, x_ref, wgu_ref, wd_ref, out_ref):
    f = wd_ref.shape[2]
    x = x_ref[...]
    wgu = wgu_ref[0]
    gu = lax.dot_general(x, wgu, (((1,), (1,)), ((), ())),
                         preferred_element_type=jnp.float32)
    g = gu[:, :f]
    u = gu[:, f:]
    h = (g * jax.nn.sigmoid(g)) * u
    h = h * w_ref[0, 0][:, None]
    wd = wd_ref[0]
    out_ref[...] = lax.dot_general(h, wd, (((1,), (1,)), ((), ())),
                                   preferred_element_type=jnp.float32)


def _run_moe(tile_expert, w_pad3, x_sorted, w_gate_up, w_down, bt):
    np_, d = x_sorted.shape
    e, f2, _ = w_gate_up.shape
    f = f2 // 2
    nt = np_ // bt
    grid_spec = pltpu.PrefetchScalarGridSpec(
        num_scalar_prefetch=1,
        grid=(nt,),
        in_specs=[
            pl.BlockSpec((1, 1, bt), lambda i, te: (i, 0, 0)),
            pl.BlockSpec((bt, d), lambda i, te: (i, 0)),
            pl.BlockSpec((1, f2, d), lambda i, te: (te[i], 0, 0)),
            pl.BlockSpec((1, d, f), lambda i, te: (te[i], 0, 0)),
        ],
        out_specs=pl.BlockSpec((bt, d), lambda i, te: (i, 0)),
    )
    return pl.pallas_call(
        _moe_body,
        grid_spec=grid_spec,
        out_shape=jax.ShapeDtypeStruct((np_, d), jnp.float32),
    )(tile_expert, w_pad3, x_sorted, w_gate_up, w_down)


def _shared_body(x_ref, wgu_ref, wd_ref, g_ref, ye_ref, out_ref):
    d = x_ref.shape[1]
    sf = wd_ref.shape[1]
    x = x_ref[...]
    wgu = wgu_ref[...]
    gu = lax.dot_general(x, wgu, (((1,), (1,)), ((), ())),
                         preferred_element_type=jnp.float32)
    g = gu[:, :sf]
    u = gu[:, sf:]
    h = (g * jax.nn.sigmoid(g)) * u
    wd = wd_ref[...]
    sh = lax.dot_general(h, wd, (((1,), (1,)), ((), ())),
                         preferred_element_type=jnp.float32)
    gate_logit = lax.dot_general(x, g_ref[...], (((1,), (1,)), ((), ())),
                                 preferred_element_type=jnp.float32)
    ye = ye_ref[...]
    out_ref[...] = jax.nn.sigmoid(gate_logit) * sh + ye[:, :d] + ye[:, d:]


def _run_shared_combine(x, sh_gate_up, sh_down, sh_gate, ye_pairs, bs):
    t, d = x.shape
    sf2 = sh_gate_up.shape[0]
    sf = sh_down.shape[1]
    return pl.pallas_call(
        _shared_body,
        grid=(t // bs,),
        in_specs=[
            pl.BlockSpec((bs, d), lambda i: (i, 0)),
            pl.BlockSpec((sf2, d), lambda i: (0, 0)),
            pl.BlockSpec((d, sf), lambda i: (0, 0)),
            pl.BlockSpec((1, d), lambda i: (0, 0)),
            pl.BlockSpec((bs, 2 * d), lambda i: (i, 0)),
        ],
        out_specs=pl.BlockSpec((bs, d), lambda i: (i, 0)),
        out_shape=jax.ShapeDtypeStruct((t, d), jnp.float32),
    )(x, sh_gate_up, sh_down, sh_gate, ye_pairs)


def _routing_metadata(topk_ids, topk_w, e, bt, nt):
    tk = topk_ids.size
    k = topk_ids.shape[1]
    np_ = nt * bt
    e_flat = topk_ids.reshape(-1)
    w_flat = topk_w.reshape(-1)
    t_flat = jnp.arange(tk, dtype=jnp.int32) // k
    oh = (e_flat[:, None] == jnp.arange(e, dtype=jnp.int32)[None, :]).astype(jnp.int32)
    incl = jnp.cumsum(oh, axis=0)
    counts = incl[-1]
    rank = jnp.take_along_axis(incl, e_flat[:, None], axis=1)[:, 0] - 1
    tiles = (counts + bt - 1) // bt
    tile_off = jnp.concatenate(
        [jnp.zeros((1,), jnp.int32), jnp.cumsum(tiles)[:-1].astype(jnp.int32)])
    slot = tile_off[e_flat] * bt + rank
    row_token = jnp.zeros((np_,), jnp.int32).at[slot].set(t_flat)
    w_pad = jnp.zeros((np_,), jnp.float32).at[slot].set(w_flat)
    ti = jnp.arange(nt, dtype=jnp.int32)
    tile_expert = jnp.sum((ti[:, None] >= tile_off[None, :]).astype(jnp.int32),
                          axis=1) - 1
    return row_token, w_pad, tile_expert, slot


def kernel(hidden_states, gate_w, w_gate_up, w_down, sh_gate_up, sh_down, sh_gate):
    b, s, d = hidden_states.shape
    e = gate_w.shape[0]
    t = b * s
    k = 2
    bt = 128
    nt = (t * k) // bt + e
    np_ = nt * bt

    x = hidden_states.reshape(t, d)

    topk_w, topk_ids = _run_router(x, gate_w, br=256)

    row_token, w_pad, tile_expert, slot = _routing_metadata(
        topk_ids, topk_w, e, bt, nt)

    x_sorted = _sc_gather(x, row_token, np_, d, chunk=32)

    w_pad3 = w_pad.reshape(nt, 1, bt)
    ye_sorted = _run_moe(tile_expert, w_pad3, x_sorted, w_gate_up, w_down, bt)

    ye_pairs = _sc_gather(ye_sorted, slot, t * k, d, chunk=32)
    ye_pairs = ye_pairs.reshape(t, k * d)

    out = _run_shared_combine(x, sh_gate_up, sh_down, sh_gate, ye_pairs, bs=256)
    return out.reshape(b, s, d)


# speedup vs baseline: 2.0406x; 1.4303x over previous
import functools

import jax
import jax.numpy as jnp
from jax import lax
from jax.experimental import pallas as pl
from jax.experimental.pallas import tpu as pltpu
from jax.experimental.pallas import tpu_sc as plsc

_SC_WORKERS = 32


def _router_body(x_ref, gw_ref, w_ref, id_ref):
    x = x_ref[...]
    gw = gw_ref[...]
    logits = lax.dot_general(x, gw, (((1,), (1,)), ((), ())),
                             preferred_element_type=jnp.float32)
    m = jnp.max(logits, axis=1, keepdims=True)
    ex = jnp.exp(logits - m)
    p = ex / jnp.sum(ex, axis=1, keepdims=True)
    ncols = p.shape[1]
    iota = lax.broadcasted_iota(jnp.int32, p.shape, 1)
    p1 = jnp.max(p, axis=1, keepdims=True)
    i1 = jnp.min(jnp.where(p >= p1, iota, ncols), axis=1, keepdims=True)
    pm = jnp.where(iota == i1, -1.0, p)
    p2 = jnp.max(pm, axis=1, keepdims=True)
    i2 = jnp.min(jnp.where(pm >= p2, iota, ncols), axis=1, keepdims=True)
    s = p1 + p2
    w_ref[...] = jnp.concatenate([p1 / s, p2 / s], axis=1)
    id_ref[...] = jnp.concatenate([i1, i2], axis=1).astype(jnp.int32)


def _run_router(x, gate_w, br):
    t, d = x.shape
    e = gate_w.shape[0]
    return pl.pallas_call(
        _router_body,
        grid=(t // br,),
        in_specs=[
            pl.BlockSpec((br, d), lambda i: (i, 0)),
            pl.BlockSpec((e, d), lambda i: (0, 0)),
        ],
        out_specs=[
            pl.BlockSpec((br, 2), lambda i: (i, 0)),
            pl.BlockSpec((br, 2), lambda i: (i, 0)),
        ],
        out_shape=[
            jax.ShapeDtypeStruct((t, 2), jnp.float32),
            jax.ShapeDtypeStruct((t, 2), jnp.int32),
        ],
    )(x, gate_w)


def _routing_metadata(topk_ids, e, bt, nt):
    e_flat = topk_ids.reshape(-1)
    oh = (e_flat[:, None] == jnp.arange(e, dtype=jnp.int32)[None, :]
          ).astype(jnp.int32)
    incl = jnp.cumsum(oh, axis=0)
    counts = incl[-1]
    rank = jnp.sum(incl * oh, axis=1) - 1
    tiles = (counts + bt - 1) // bt
    tile_off = jnp.concatenate(
        [jnp.zeros((1,), jnp.int32), jnp.cumsum(tiles)[:-1].astype(jnp.int32)])
    row_base = tile_off * bt
    slot = jnp.sum(oh * row_base[None, :], axis=1) + rank
    ti = jnp.arange(nt, dtype=jnp.int32)
    tile_expert = jnp.sum((ti[:, None] >= tile_off[None, :]).astype(jnp.int32),
                          axis=1) - 1
    return slot.astype(jnp.int32), tile_expert


def _sc_dispatch(x, slot0, slot1, np_rows, chunk):
    t, d = x.shape
    tpw = t // _SC_WORKERS
    n_ch = tpw // chunk
    mesh = plsc.VectorSubcoreMesh(core_axis_name="c", subcore_axis_name="s")

    @functools.partial(
        pl.kernel,
        mesh=mesh,
        out_type=jax.ShapeDtypeStruct((np_rows, d), jnp.float32),
        scratch_types=[
            pltpu.VMEM((2, chunk, d), jnp.float32),
            pltpu.VMEM((2, chunk), jnp.int32),
            pltpu.VMEM((2, chunk), jnp.int32),
            pltpu.SemaphoreType.DMA,
            pltpu.SemaphoreType.DMA,
        ],
    )
    def dk(x_hbm, s0_hbm, s1_hbm, out_hbm, rows_v, i0_v, i1_v, ws0, ws1):
        wid = lax.axis_index("s") * 2 + lax.axis_index("c")
        base = wid * tpw
        wsems = (ws0, ws1)
        pending = {}
        for c in range(n_ch):
            b = c % 2
            if c >= 2:
                pending[(c - 2, 0)].wait()
                pending[(c - 2, 1)].wait()
            off = base + c * chunk
            pltpu.sync_copy(s0_hbm.at[pl.ds(off, chunk)], i0_v.at[b])
            pltpu.sync_copy(s1_hbm.at[pl.ds(off, chunk)], i1_v.at[b])
            pltpu.sync_copy(x_hbm.at[pl.ds(off, chunk)], rows_v.at[b])
            pending[(c, 0)] = pltpu.async_copy(
                rows_v.at[b], out_hbm.at[i0_v.at[b]], wsems[b])
            pending[(c, 1)] = pltpu.async_copy(
                rows_v.at[b], out_hbm.at[i1_v.at[b]], wsems[b])
        for c in range(max(0, n_ch - 2), n_ch):
            pending[(c, 0)].wait()
            pending[(c, 1)].wait()

    return dk(x, slot0, slot1)


def _sc_gather(table, idx, nrows, chunk):
    d = table.shape[1]
    b_per_w = nrows // _SC_WORKERS
    n_ch = b_per_w // chunk
    mesh = plsc.VectorSubcoreMesh(core_axis_name="c", subcore_axis_name="s")

    @functools.partial(
        pl.kernel,
        mesh=mesh,
        out_type=jax.ShapeDtypeStruct((nrows, d), jnp.float32),
        scratch_types=[
            pltpu.VMEM((2, chunk, d), jnp.float32),
            pltpu.VMEM((2, chunk), jnp.int32),
            pltpu.SemaphoreType.DMA,
            pltpu.SemaphoreType.DMA,
            pltpu.SemaphoreType.DMA,
        ],
    )
    def gk(table_hbm, idx_hbm, out_hbm, rows_v, idx_v, gs, ws0, ws1):
        wid = lax.axis_index("s") * 2 + lax.axis_index("c")
        base = wid * b_per_w
        wsems = (ws0, ws1)
        pending = {}
        for c in range(n_ch):
            b = c % 2
            if c >= 2:
                pending[c - 2].wait()
            off = base + c * chunk
            pltpu.sync_copy(idx_hbm.at[pl.ds(off, chunk)], idx_v.at[b])
            pltpu.async_copy(table_hbm.at[idx_v.at[b]], rows_v.at[b], gs).wait()
            pending[c] = pltpu.async_copy(
                rows_v.at[b], out_hbm.at[pl.ds(off, chunk)], wsems[b])
        for c in range(max(0, n_ch - 2), n_ch):
            pending[c].wait()

    return gk(table, idx)


def _moe_body(te_ref, x_ref, wgu_ref, wd_ref, out_ref):
    f = wd_ref.shape[2]
    x = x_ref[...]
    wgu = wgu_ref[0]
    gu = lax.dot_general(x, wgu, (((1,), (1,)), ((), ())),
                         preferred_element_type=jnp.float32)
    g = gu[:, :f]
    u = gu[:, f:]
    h = (g * jax.nn.sigmoid(g)) * u
    wd = wd_ref[0]
    out_ref[...] = lax.dot_general(h, wd, (((1,), (1,)), ((), ())),
                                   preferred_element_type=jnp.float32)


def _run_moe(tile_expert, x_sorted, w_gate_up, w_down, bt):
    np_, d = x_sorted.shape
    e, f2, _ = w_gate_up.shape
    f = f2 // 2
    nt = np_ // bt
    grid_spec = pltpu.PrefetchScalarGridSpec(
        num_scalar_prefetch=1,
        grid=(nt,),
        in_specs=[
            pl.BlockSpec((bt, d), lambda i, te: (i, 0)),
            pl.BlockSpec((1, f2, d), lambda i, te: (te[i], 0, 0)),
            pl.BlockSpec((1, d, f), lambda i, te: (te[i], 0, 0)),
        ],
        out_specs=pl.BlockSpec((bt, d), lambda i, te: (i, 0)),
    )
    return pl.pallas_call(
        _moe_body,
        grid_spec=grid_spec,
        out_shape=jax.ShapeDtypeStruct((np_, d), jnp.float32),
    )(tile_expert, x_sorted, w_gate_up, w_down)


def _shared_body(x_ref, wgu_ref, wd_ref, g_ref, out_ref):
    sf = wd_ref.shape[1]
    x = x_ref[...]
    wgu = wgu_ref[...]
    gu = lax.dot_general(x, wgu, (((1,), (1,)), ((), ())),
                         preferred_element_type=jnp.float32)
    g = gu[:, :sf]
    u = gu[:, sf:]
    h = (g * jax.nn.sigmoid(g)) * u
    wd = wd_ref[...]
    sh = lax.dot_general(h, wd, (((1,), (1,)), ((), ())),
                         preferred_element_type=jnp.float32)
    gate_logit = lax.dot_general(x, g_ref[...], (((1,), (1,)), ((), ())),
                                 preferred_element_type=jnp.float32)
    out_ref[...] = jax.nn.sigmoid(gate_logit) * sh


def _run_shared(x, sh_gate_up, sh_down, sh_gate, bs):
    t, d = x.shape
    sf2 = sh_gate_up.shape[0]
    sf = sh_down.shape[1]
    return pl.pallas_call(
        _shared_body,
        grid=(t // bs,),
        in_specs=[
            pl.BlockSpec((bs, d), lambda i: (i, 0)),
            pl.BlockSpec((sf2, d), lambda i: (0, 0)),
            pl.BlockSpec((d, sf), lambda i: (0, 0)),
            pl.BlockSpec((1, d), lambda i: (0, 0)),
        ],
        out_specs=pl.BlockSpec((bs, d), lambda i: (i, 0)),
        out_shape=jax.ShapeDtypeStruct((t, d), jnp.float32),
    )(x, sh_gate_up, sh_down, sh_gate)


def _combine_body(sh_ref, ye_ref, w_ref, out_ref):
    bs, d = sh_ref.shape
    ye = ye_ref[...]
    ye3 = ye.reshape(bs, 2, d)
    w = w_ref[...]
    out_ref[...] = (sh_ref[...]
                    + w[:, 0:1] * ye3[:, 0, :]
                    + w[:, 1:2] * ye3[:, 1, :])


def _run_combine(sh, ye_pairs, topk_w, bs):
    t, d = sh.shape
    return pl.pallas_call(
        _combine_body,
        grid=(t // bs,),
        in_specs=[
            pl.BlockSpec((bs, d), lambda i: (i, 0)),
            pl.BlockSpec((2 * bs, d), lambda i: (i, 0)),
            pl.BlockSpec((bs, 2), lambda i: (i, 0)),
        ],
        out_specs=pl.BlockSpec((bs, d), lambda i: (i, 0)),
        out_shape=jax.ShapeDtypeStruct((t, d), jnp.float32),
    )(sh, ye_pairs, topk_w)


def kernel(hidden_states, gate_w, w_gate_up, w_down, sh_gate_up, sh_down, sh_gate):
    b, s, d = hidden_states.shape
    e = gate_w.shape[0]
    t = b * s
    k = 2
    bt = 128
    nt = (t * k) // bt + e
    np_ = nt * bt

    x = hidden_states.reshape(t, d)

    topk_w, topk_ids = _run_router(x, gate_w, br=256)
    slot, tile_expert = _routing_metadata(topk_ids, e, bt, nt)
    slot2 = slot.reshape(t, k)
    slot0 = slot2[:, 0]
    slot1 = slot2[:, 1]

    x_sorted = _sc_dispatch(x, slot0, slot1, np_, chunk=16)
    ye_sorted = _run_moe(tile_expert, x_sorted, w_gate_up, w_down, bt)
    ye_pairs = _sc_gather(ye_sorted, slot, t * k, chunk=16)

    sh = _run_shared(x, sh_gate_up, sh_down, sh_gate, bs=256)
    out = _run_combine(sh, ye_pairs, topk_w, bs=256)
    return out.reshape(b, s, d)


# speedup vs baseline: 2.2247x; 1.0902x over previous
import functools

import jax
import jax.numpy as jnp
from jax import lax
from jax.experimental import pallas as pl
from jax.experimental.pallas import tpu as pltpu
from jax.experimental.pallas import tpu_sc as plsc

_SC_WORKERS = 32


def _router_body(x_ref, gw_ref, w_ref, id_ref):
    x = x_ref[...]
    gw = gw_ref[...]
    logits = lax.dot_general(x, gw, (((1,), (1,)), ((), ())),
                             preferred_element_type=jnp.float32)
    m = jnp.max(logits, axis=1, keepdims=True)
    ex = jnp.exp(logits - m)
    p = ex / jnp.sum(ex, axis=1, keepdims=True)
    ncols = p.shape[1]
    iota = lax.broadcasted_iota(jnp.int32, p.shape, 1)
    p1 = jnp.max(p, axis=1, keepdims=True)
    i1 = jnp.min(jnp.where(p >= p1, iota, ncols), axis=1, keepdims=True)
    pm = jnp.where(iota == i1, -1.0, p)
    p2 = jnp.max(pm, axis=1, keepdims=True)
    i2 = jnp.min(jnp.where(pm >= p2, iota, ncols), axis=1, keepdims=True)
    s = p1 + p2
    w_ref[...] = jnp.concatenate([p1 / s, p2 / s], axis=1)
    id_ref[...] = jnp.concatenate([i1, i2], axis=1).astype(jnp.int32)


def _run_router(x, gate_w, br):
    t, d = x.shape
    e = gate_w.shape[0]
    return pl.pallas_call(
        _router_body,
        grid=(t // br,),
        in_specs=[
            pl.BlockSpec((br, d), lambda i: (i, 0)),
            pl.BlockSpec((e, d), lambda i: (0, 0)),
        ],
        out_specs=[
            pl.BlockSpec((br, 2), lambda i: (i, 0)),
            pl.BlockSpec((br, 2), lambda i: (i, 0)),
        ],
        out_shape=[
            jax.ShapeDtypeStruct((t, 2), jnp.float32),
            jax.ShapeDtypeStruct((t, 2), jnp.int32),
        ],
    )(x, gate_w)


def _routing_metadata(topk_ids, e, bt, nt):
    e_flat = topk_ids.reshape(-1)
    oh = (e_flat[:, None] == jnp.arange(e, dtype=jnp.int32)[None, :]
          ).astype(jnp.int32)
    incl = jnp.cumsum(oh, axis=0)
    counts = incl[-1]
    rank = jnp.sum(incl * oh, axis=1) - 1
    tiles = (counts + bt - 1) // bt
    tile_off = jnp.concatenate(
        [jnp.zeros((1,), jnp.int32), jnp.cumsum(tiles)[:-1].astype(jnp.int32)])
    row_base = tile_off * bt
    slot = jnp.sum(oh * row_base[None, :], axis=1) + rank
    ti = jnp.arange(nt, dtype=jnp.int32)
    tile_expert = jnp.sum((ti[:, None] >= tile_off[None, :]).astype(jnp.int32),
                          axis=1) - 1
    live_tiles = jnp.sum(tiles).astype(jnp.int32).reshape(1)
    return slot.astype(jnp.int32), tile_expert, live_tiles


def _sc_dispatch(x, slot0, slot1, np_rows, chunk):
    t, d = x.shape
    tpw = t // _SC_WORKERS
    n_ch = tpw // chunk
    mesh = plsc.VectorSubcoreMesh(core_axis_name="c", subcore_axis_name="s")

    @functools.partial(
        pl.kernel,
        mesh=mesh,
        out_type=jax.ShapeDtypeStruct((np_rows, d), jnp.float32),
        scratch_types=[
            pltpu.VMEM((2, chunk, d), jnp.float32),
            pltpu.VMEM((2, chunk), jnp.int32),
            pltpu.VMEM((2, chunk), jnp.int32),
            pltpu.SemaphoreType.DMA,
            pltpu.SemaphoreType.DMA,
        ],
    )
    def dk(x_hbm, s0_hbm, s1_hbm, out_hbm, rows_v, i0_v, i1_v, ws0, ws1):
        wid = lax.axis_index("s") * 2 + lax.axis_index("c")
        base = wid * tpw
        wsems = (ws0, ws1)
        pending = {}
        for c in range(n_ch):
            b = c % 2
            if c >= 2:
                pending[(c - 2, 0)].wait()
                pending[(c - 2, 1)].wait()
            off = base + c * chunk
            pltpu.sync_copy(s0_hbm.at[pl.ds(off, chunk)], i0_v.at[b])
            pltpu.sync_copy(s1_hbm.at[pl.ds(off, chunk)], i1_v.at[b])
            pltpu.sync_copy(x_hbm.at[pl.ds(off, chunk)], rows_v.at[b])
            pending[(c, 0)] = pltpu.async_copy(
                rows_v.at[b], out_hbm.at[i0_v.at[b]], wsems[b])
            pending[(c, 1)] = pltpu.async_copy(
                rows_v.at[b], out_hbm.at[i1_v.at[b]], wsems[b])
        for c in range(max(0, n_ch - 2), n_ch):
            pending[(c, 0)].wait()
            pending[(c, 1)].wait()

    return dk(x, slot0, slot1)


def _sc_gather(table, idx, nrows, chunk):
    d = table.shape[1]
    b_per_w = nrows // _SC_WORKERS
    n_ch = b_per_w // chunk
    mesh = plsc.VectorSubcoreMesh(core_axis_name="c", subcore_axis_name="s")

    @functools.partial(
        pl.kernel,
        mesh=mesh,
        out_type=jax.ShapeDtypeStruct((nrows, d), jnp.float32),
        scratch_types=[
            pltpu.VMEM((2, chunk, d), jnp.float32),
            pltpu.VMEM((2, chunk), jnp.int32),
            pltpu.SemaphoreType.DMA,
            pltpu.SemaphoreType.DMA,
            pltpu.SemaphoreType.DMA,
        ],
    )
    def gk(table_hbm, idx_hbm, out_hbm, rows_v, idx_v, gs, ws0, ws1):
        wid = lax.axis_index("s") * 2 + lax.axis_index("c")
        base = wid * b_per_w
        wsems = (ws0, ws1)
        pending = {}
        for c in range(n_ch):
            b = c % 2
            if c >= 2:
                pending[c - 2].wait()
            off = base + c * chunk
            pltpu.sync_copy(idx_hbm.at[pl.ds(off, chunk)], idx_v.at[b])
            pltpu.async_copy(table_hbm.at[idx_v.at[b]], rows_v.at[b], gs).wait()
            pending[c] = pltpu.async_copy(
                rows_v.at[b], out_hbm.at[pl.ds(off, chunk)], wsems[b])
        for c in range(max(0, n_ch - 2), n_ch):
            pending[c].wait()

    return gk(table, idx)


def _moe_body(te_ref, lc_ref, x_ref, wgu_ref, wd_ref, out_ref):
    f = wd_ref.shape[2]
    hf = f // 2

    @pl.when(pl.program_id(0) < lc_ref[0])
    def _():
        x = x_ref[...]
        acc = None
        for j in range(2):
            wg = wgu_ref[0, j * hf:(j + 1) * hf, :]
            wu = wgu_ref[0, f + j * hf:f + (j + 1) * hf, :]
            gj = lax.dot_general(x, wg, (((1,), (1,)), ((), ())),
                                 preferred_element_type=jnp.float32)
            uj = lax.dot_general(x, wu, (((1,), (1,)), ((), ())),
                                 preferred_element_type=jnp.float32)
            hj = (gj * jax.nn.sigmoid(gj)) * uj
            wdj = wd_ref[0, :, j * hf:(j + 1) * hf]
            pj = lax.dot_general(hj, wdj, (((1,), (1,)), ((), ())),
                                 preferred_element_type=jnp.float32)
            acc = pj if acc is None else acc + pj
        out_ref[...] = acc


def _run_moe(tile_expert, live_tiles, x_sorted, w_gate_up, w_down, bt):
    np_, d = x_sorted.shape
    e, f2, _ = w_gate_up.shape
    f = f2 // 2
    nt = np_ // bt
    grid_spec = pltpu.PrefetchScalarGridSpec(
        num_scalar_prefetch=2,
        grid=(nt,),
        in_specs=[
            pl.BlockSpec((bt, d), lambda i, te, lc: (i, 0)),
            pl.BlockSpec((1, f2, d), lambda i, te, lc: (te[i], 0, 0)),
            pl.BlockSpec((1, d, f), lambda i, te, lc: (te[i], 0, 0)),
        ],
        out_specs=pl.BlockSpec((bt, d), lambda i, te, lc: (i, 0)),
    )
    return pl.pallas_call(
        _moe_body,
        grid_spec=grid_spec,
        out_shape=jax.ShapeDtypeStruct((np_, d), jnp.float32),
    )(tile_expert, live_tiles, x_sorted, w_gate_up, w_down)


def _shared_body(x_ref, wgu_ref, wd_ref, g_ref, out_ref):
    sf = wd_ref.shape[1]
    x = x_ref[...]
    wgu = wgu_ref[...]
    gu = lax.dot_general(x, wgu, (((1,), (1,)), ((), ())),
                         preferred_element_type=jnp.float32)
    g = gu[:, :sf]
    u = gu[:, sf:]
    h = (g * jax.nn.sigmoid(g)) * u
    wd = wd_ref[...]
    sh = lax.dot_general(h, wd, (((1,), (1,)), ((), ())),
                         preferred_element_type=jnp.float32)
    gate_logit = lax.dot_general(x, g_ref[...], (((1,), (1,)), ((), ())),
                                 preferred_element_type=jnp.float32)
    out_ref[...] = jax.nn.sigmoid(gate_logit) * sh


def _run_shared(x, sh_gate_up, sh_down, sh_gate, bs):
    t, d = x.shape
    sf2 = sh_gate_up.shape[0]
    sf = sh_down.shape[1]
    return pl.pallas_call(
        _shared_body,
        grid=(t // bs,),
        in_specs=[
            pl.BlockSpec((bs, d), lambda i: (i, 0)),
            pl.BlockSpec((sf2, d), lambda i: (0, 0)),
            pl.BlockSpec((d, sf), lambda i: (0, 0)),
            pl.BlockSpec((1, d), lambda i: (0, 0)),
        ],
        out_specs=pl.BlockSpec((bs, d), lambda i: (i, 0)),
        out_shape=jax.ShapeDtypeStruct((t, d), jnp.float32),
    )(x, sh_gate_up, sh_down, sh_gate)


def _combine_body(sh_ref, ye_ref, w_ref, out_ref):
    bs, d = sh_ref.shape
    ye = ye_ref[...]
    ye3 = ye.reshape(bs, 2, d)
    w = w_ref[...]
    out_ref[...] = (sh_ref[...]
                    + w[:, 0:1] * ye3[:, 0, :]
                    + w[:, 1:2] * ye3[:, 1, :])


def _run_combine(sh, ye_pairs, topk_w, bs):
    t, d = sh.shape
    return pl.pallas_call(
        _combine_body,
        grid=(t // bs,),
        in_specs=[
            pl.BlockSpec((bs, d), lambda i: (i, 0)),
            pl.BlockSpec((2 * bs, d), lambda i: (i, 0)),
            pl.BlockSpec((bs, 2), lambda i: (i, 0)),
        ],
        out_specs=pl.BlockSpec((bs, d), lambda i: (i, 0)),
        out_shape=jax.ShapeDtypeStruct((t, d), jnp.float32),
    )(sh, ye_pairs, topk_w)


def kernel(hidden_states, gate_w, w_gate_up, w_down, sh_gate_up, sh_down, sh_gate):
    b, s, d = hidden_states.shape
    e = gate_w.shape[0]
    t = b * s
    k = 2
    bt = 128
    nt = (t * k) // bt + e
    np_ = nt * bt

    x = hidden_states.reshape(t, d)

    topk_w, topk_ids = _run_router(x, gate_w, br=256)
    slot, tile_expert, live_tiles = _routing_metadata(topk_ids, e, bt, nt)
    slot2 = slot.reshape(t, k)
    slot0 = slot2[:, 0]
    slot1 = slot2[:, 1]

    x_sorted = _sc_dispatch(x, slot0, slot1, np_, chunk=16)
    ye_sorted = _run_moe(tile_expert, live_tiles, x_sorted, w_gate_up, w_down, bt)
    ye_pairs = _sc_gather(ye_sorted, slot, t * k, chunk=16)

    sh = _run_shared(x, sh_gate_up, sh_down, sh_gate, bs=256)
    out = _run_combine(sh, ye_pairs, topk_w, bs=256)
    return out.reshape(b, s, d)


# speedup vs baseline: 2.2264x; 1.0007x over previous
import functools

import jax
import jax.numpy as jnp
from jax import lax
from jax.experimental import pallas as pl
from jax.experimental.pallas import tpu as pltpu
from jax.experimental.pallas import tpu_sc as plsc

_SC_WORKERS = 32


def _router_body(x_ref, gw_ref, w_ref, id_ref):
    x = x_ref[...]
    gw = gw_ref[...]
    logits = lax.dot_general(x, gw, (((1,), (1,)), ((), ())),
                             preferred_element_type=jnp.float32)
    m = jnp.max(logits, axis=1, keepdims=True)
    ex = jnp.exp(logits - m)
    p = ex / jnp.sum(ex, axis=1, keepdims=True)
    ncols = p.shape[1]
    iota = lax.broadcasted_iota(jnp.int32, p.shape, 1)
    p1 = jnp.max(p, axis=1, keepdims=True)
    i1 = jnp.min(jnp.where(p >= p1, iota, ncols), axis=1, keepdims=True)
    pm = jnp.where(iota == i1, -1.0, p)
    p2 = jnp.max(pm, axis=1, keepdims=True)
    i2 = jnp.min(jnp.where(pm >= p2, iota, ncols), axis=1, keepdims=True)
    s = p1 + p2
    w_ref[...] = jnp.concatenate([p1 / s, p2 / s], axis=1)
    id_ref[...] = jnp.concatenate([i1, i2], axis=1).astype(jnp.int32)


def _run_router(x, gate_w, br):
    t, d = x.shape
    e = gate_w.shape[0]
    return pl.pallas_call(
        _router_body,
        grid=(t // br,),
        in_specs=[
            pl.BlockSpec((br, d), lambda i: (i, 0)),
            pl.BlockSpec((e, d), lambda i: (0, 0)),
        ],
        out_specs=[
            pl.BlockSpec((br, 2), lambda i: (i, 0)),
            pl.BlockSpec((br, 2), lambda i: (i, 0)),
        ],
        out_shape=[
            jax.ShapeDtypeStruct((t, 2), jnp.float32),
            jax.ShapeDtypeStruct((t, 2), jnp.int32),
        ],
    )(x, gate_w)


def _routing_metadata(topk_ids, e, bt, nt):
    e_flat = topk_ids.reshape(-1)
    oh = (e_flat[:, None] == jnp.arange(e, dtype=jnp.int32)[None, :]
          ).astype(jnp.int32)
    incl = jnp.cumsum(oh, axis=0)
    counts = incl[-1]
    rank = jnp.sum(incl * oh, axis=1) - 1
    tiles = (counts + bt - 1) // bt
    tile_off = jnp.concatenate(
        [jnp.zeros((1,), jnp.int32), jnp.cumsum(tiles)[:-1].astype(jnp.int32)])
    row_base = tile_off * bt
    slot = jnp.sum(oh * row_base[None, :], axis=1) + rank
    ti = jnp.arange(nt, dtype=jnp.int32)
    tile_expert = jnp.sum((ti[:, None] >= tile_off[None, :]).astype(jnp.int32),
                          axis=1) - 1
    live_tiles = jnp.sum(tiles).astype(jnp.int32).reshape(1)
    return slot.astype(jnp.int32), tile_expert, live_tiles


def _sc_dispatch(x, slot0, slot1, np_rows, chunk):
    t, d = x.shape
    tpw = t // _SC_WORKERS
    n_ch = tpw // chunk
    mesh = plsc.VectorSubcoreMesh(core_axis_name="c", subcore_axis_name="s")

    @functools.partial(
        pl.kernel,
        mesh=mesh,
        out_type=jax.ShapeDtypeStruct((np_rows, d), jnp.float32),
        scratch_types=[
            pltpu.VMEM((2, chunk, d), jnp.float32),
            pltpu.VMEM((2, chunk), jnp.int32),
            pltpu.VMEM((2, chunk), jnp.int32),
            pltpu.SemaphoreType.DMA,
            pltpu.SemaphoreType.DMA,
        ],
    )
    def dk(x_hbm, s0_hbm, s1_hbm, out_hbm, rows_v, i0_v, i1_v, ws0, ws1):
        wid = lax.axis_index("s") * 2 + lax.axis_index("c")
        base = wid * tpw
        wsems = (ws0, ws1)
        pending = {}
        for c in range(n_ch):
            b = c % 2
            if c >= 2:
                pending[(c - 2, 0)].wait()
                pending[(c - 2, 1)].wait()
            off = base + c * chunk
            pltpu.sync_copy(s0_hbm.at[pl.ds(off, chunk)], i0_v.at[b])
            pltpu.sync_copy(s1_hbm.at[pl.ds(off, chunk)], i1_v.at[b])
            pltpu.sync_copy(x_hbm.at[pl.ds(off, chunk)], rows_v.at[b])
            pending[(c, 0)] = pltpu.async_copy(
                rows_v.at[b], out_hbm.at[i0_v.at[b]], wsems[b])
            pending[(c, 1)] = pltpu.async_copy(
                rows_v.at[b], out_hbm.at[i1_v.at[b]], wsems[b])
        for c in range(max(0, n_ch - 2), n_ch):
            pending[(c, 0)].wait()
            pending[(c, 1)].wait()

    return dk(x, slot0, slot1)


def _sc_gather(table, idx, nrows, chunk):
    d = table.shape[1]
    b_per_w = nrows // _SC_WORKERS
    n_ch = b_per_w // chunk
    mesh = plsc.VectorSubcoreMesh(core_axis_name="c", subcore_axis_name="s")

    @functools.partial(
        pl.kernel,
        mesh=mesh,
        out_type=jax.ShapeDtypeStruct((nrows, d), jnp.float32),
        scratch_types=[
            pltpu.VMEM((2, chunk, d), jnp.float32),
            pltpu.VMEM((2, chunk), jnp.int32),
            pltpu.SemaphoreType.DMA,
            pltpu.SemaphoreType.DMA,
            pltpu.SemaphoreType.DMA,
        ],
    )
    def gk(table_hbm, idx_hbm, out_hbm, rows_v, idx_v, gs, ws0, ws1):
        wid = lax.axis_index("s") * 2 + lax.axis_index("c")
        base = wid * b_per_w
        wsems = (ws0, ws1)
        pending = {}
        for c in range(n_ch):
            b = c % 2
            if c >= 2:
                pending[c - 2].wait()
            off = base + c * chunk
            pltpu.sync_copy(idx_hbm.at[pl.ds(off, chunk)], idx_v.at[b])
            pltpu.async_copy(table_hbm.at[idx_v.at[b]], rows_v.at[b], gs).wait()
            pending[c] = pltpu.async_copy(
                rows_v.at[b], out_hbm.at[pl.ds(off, chunk)], wsems[b])
        for c in range(max(0, n_ch - 2), n_ch):
            pending[c].wait()

    return gk(table, idx)


def _moe_body(te_ref, lc_ref, x_ref, wgu_ref, wd_ref, out_ref):
    f = wd_ref.shape[2]
    hf = f // 2

    @pl.when(pl.program_id(0) < lc_ref[0])
    def _():
        x = x_ref[...]
        acc = None
        for j in range(2):
            wg = wgu_ref[0, j * hf:(j + 1) * hf, :]
            wu = wgu_ref[0, f + j * hf:f + (j + 1) * hf, :]
            gj = lax.dot_general(x, wg, (((1,), (1,)), ((), ())),
                                 preferred_element_type=jnp.float32)
            uj = lax.dot_general(x, wu, (((1,), (1,)), ((), ())),
                                 preferred_element_type=jnp.float32)
            hj = (gj * jax.nn.sigmoid(gj)) * uj
            wdj = wd_ref[0, :, j * hf:(j + 1) * hf]
            pj = lax.dot_general(hj, wdj, (((1,), (1,)), ((), ())),
                                 preferred_element_type=jnp.float32)
            acc = pj if acc is None else acc + pj
        out_ref[...] = acc


def _run_moe(tile_expert, live_tiles, x_sorted, w_gate_up, w_down, bt):
    np_, d = x_sorted.shape
    e, f2, _ = w_gate_up.shape
    f = f2 // 2
    nt = np_ // bt
    grid_spec = pltpu.PrefetchScalarGridSpec(
        num_scalar_prefetch=2,
        grid=(nt,),
        in_specs=[
            pl.BlockSpec((bt, d), lambda i, te, lc: (i, 0)),
            pl.BlockSpec((1, f2, d), lambda i, te, lc: (te[i], 0, 0)),
            pl.BlockSpec((1, d, f), lambda i, te, lc: (te[i], 0, 0)),
        ],
        out_specs=pl.BlockSpec((bt, d), lambda i, te, lc: (i, 0)),
    )
    return pl.pallas_call(
        _moe_body,
        grid_spec=grid_spec,
        out_shape=jax.ShapeDtypeStruct((np_, d), jnp.float32),
    )(tile_expert, live_tiles, x_sorted, w_gate_up, w_down)


def _shared_body(x_ref, wgu_ref, wd_ref, g_ref, out_ref):
    sf = wd_ref.shape[1]
    x = x_ref[...]
    wgu = wgu_ref[...]
    gu = lax.dot_general(x, wgu, (((1,), (1,)), ((), ())),
                         preferred_element_type=jnp.float32)
    g = gu[:, :sf]
    u = gu[:, sf:]
    h = (g * jax.nn.sigmoid(g)) * u
    wd = wd_ref[...]
    sh = lax.dot_general(h, wd, (((1,), (1,)), ((), ())),
                         preferred_element_type=jnp.float32)
    gate_logit = lax.dot_general(x, g_ref[...], (((1,), (1,)), ((), ())),
                                 preferred_element_type=jnp.float32)
    out_ref[...] = jax.nn.sigmoid(gate_logit) * sh


def _run_shared(x, sh_gate_up, sh_down, sh_gate, bs):
    t, d = x.shape
    sf2 = sh_gate_up.shape[0]
    sf = sh_down.shape[1]
    return pl.pallas_call(
        _shared_body,
        grid=(t // bs,),
        in_specs=[
            pl.BlockSpec((bs, d), lambda i: (i, 0)),
            pl.BlockSpec((sf2, d), lambda i: (0, 0)),
            pl.BlockSpec((d, sf), lambda i: (0, 0)),
            pl.BlockSpec((1, d), lambda i: (0, 0)),
        ],
        out_specs=pl.BlockSpec((bs, d), lambda i: (i, 0)),
        out_shape=jax.ShapeDtypeStruct((t, d), jnp.float32),
    )(x, sh_gate_up, sh_down, sh_gate)


def _combine_body(sh_ref, ye_ref, w_ref, out_ref):
    bs, d = sh_ref.shape
    ye = ye_ref[...]
    ye3 = ye.reshape(bs, 2, d)
    w = w_ref[...]
    out_ref[...] = (sh_ref[...]
                    + w[:, 0:1] * ye3[:, 0, :]
                    + w[:, 1:2] * ye3[:, 1, :])


def _run_combine(sh, ye_pairs, topk_w, bs):
    t, d = sh.shape
    return pl.pallas_call(
        _combine_body,
        grid=(t // bs,),
        in_specs=[
            pl.BlockSpec((bs, d), lambda i: (i, 0)),
            pl.BlockSpec((2 * bs, d), lambda i: (i, 0)),
            pl.BlockSpec((bs, 2), lambda i: (i, 0)),
        ],
        out_specs=pl.BlockSpec((bs, d), lambda i: (i, 0)),
        out_shape=jax.ShapeDtypeStruct((t, d), jnp.float32),
    )(sh, ye_pairs, topk_w)


def kernel(hidden_states, gate_w, w_gate_up, w_down, sh_gate_up, sh_down, sh_gate):
    b, s, d = hidden_states.shape
    e = gate_w.shape[0]
    t = b * s
    k = 2
    bt = 128
    nt = (t * k) // bt + e
    np_ = nt * bt

    x = hidden_states.reshape(t, d)

    topk_w, topk_ids = _run_router(x, gate_w, br=256)
    slot, tile_expert, live_tiles = _routing_metadata(topk_ids, e, bt, nt)
    slot2 = slot.reshape(t, k)
    slot0 = slot2[:, 0]
    slot1 = slot2[:, 1]

    x_sorted = _sc_dispatch(x, slot0, slot1, np_, chunk=16)
    sh = _run_shared(x, sh_gate_up, sh_down, sh_gate, bs=256)
    ye_sorted = _run_moe(tile_expert, live_tiles, x_sorted, w_gate_up, w_down, bt)
    ye_pairs = _sc_gather(ye_sorted, slot, t * k, chunk=16)

    out = _run_combine(sh, ye_pairs, topk_w, bs=256)
    return out.reshape(b, s, d)


# speedup vs baseline: 2.7322x; 1.2272x over previous
import functools

import jax
import jax.numpy as jnp
from jax import lax
from jax.experimental import pallas as pl
from jax.experimental.pallas import tpu as pltpu
from jax.experimental.pallas import tpu_sc as plsc

_SC_WORKERS = 32


def _router_body(x_ref, gw_ref, w_ref, id_ref):
    x = x_ref[...]
    gw = gw_ref[...]
    logits = lax.dot_general(x, gw, (((1,), (1,)), ((), ())),
                             preferred_element_type=jnp.float32)
    m = jnp.max(logits, axis=1, keepdims=True)
    ex = jnp.exp(logits - m)
    p = ex / jnp.sum(ex, axis=1, keepdims=True)
    ncols = p.shape[1]
    iota = lax.broadcasted_iota(jnp.int32, p.shape, 1)
    p1 = jnp.max(p, axis=1, keepdims=True)
    i1 = jnp.min(jnp.where(p >= p1, iota, ncols), axis=1, keepdims=True)
    pm = jnp.where(iota == i1, -1.0, p)
    p2 = jnp.max(pm, axis=1, keepdims=True)
    i2 = jnp.min(jnp.where(pm >= p2, iota, ncols), axis=1, keepdims=True)
    s = p1 + p2
    w_ref[...] = jnp.concatenate([p1 / s, p2 / s], axis=1)
    id_ref[...] = jnp.concatenate([i1, i2], axis=1).astype(jnp.int32)


def _run_router(x, gate_w, br):
    t, d = x.shape
    e = gate_w.shape[0]
    return pl.pallas_call(
        _router_body,
        grid=(t // br,),
        in_specs=[
            pl.BlockSpec((br, d), lambda i: (i, 0)),
            pl.BlockSpec((e, d), lambda i: (0, 0)),
        ],
        out_specs=[
            pl.BlockSpec((br, 2), lambda i: (i, 0)),
            pl.BlockSpec((br, 2), lambda i: (i, 0)),
        ],
        out_shape=[
            jax.ShapeDtypeStruct((t, 2), jnp.float32),
            jax.ShapeDtypeStruct((t, 2), jnp.int32),
        ],
    )(x, gate_w)


def _routing_metadata(topk_ids, e, bt, nt):
    e_flat = topk_ids.reshape(-1)
    oh = (e_flat[:, None] == jnp.arange(e, dtype=jnp.int32)[None, :]
          ).astype(jnp.int32)
    incl = jnp.cumsum(oh, axis=0)
    counts = incl[-1]
    rank = jnp.sum(incl * oh, axis=1) - 1
    tiles = (counts + bt - 1) // bt
    tile_off = jnp.concatenate(
        [jnp.zeros((1,), jnp.int32), jnp.cumsum(tiles)[:-1].astype(jnp.int32)])
    row_base = tile_off * bt
    slot = jnp.sum(oh * row_base[None, :], axis=1) + rank
    ti = jnp.arange(nt, dtype=jnp.int32)
    tile_expert = jnp.sum((ti[:, None] >= tile_off[None, :]).astype(jnp.int32),
                          axis=1) - 1
    live_tiles = jnp.sum(tiles).astype(jnp.int32).reshape(1)
    return slot.astype(jnp.int32), tile_expert, live_tiles


def _sc_dispatch(x, slot0, slot1, np_rows, chunk):
    t, d = x.shape
    tpw = t // _SC_WORKERS
    n_ch = tpw // chunk
    mesh = plsc.VectorSubcoreMesh(core_axis_name="c", subcore_axis_name="s")

    @functools.partial(
        pl.kernel,
        mesh=mesh,
        out_type=jax.ShapeDtypeStruct((np_rows, d), jnp.float32),
        scratch_types=[
            pltpu.VMEM((2, chunk, d), jnp.float32),
            pltpu.VMEM((2, chunk), jnp.int32),
            pltpu.VMEM((2, chunk), jnp.int32),
            pltpu.SemaphoreType.DMA,
            pltpu.SemaphoreType.DMA,
        ],
    )
    def dk(x_hbm, s0_hbm, s1_hbm, out_hbm, rows_v, i0_v, i1_v, ws0, ws1):
        wid = lax.axis_index("s") * 2 + lax.axis_index("c")
        base = wid * tpw
        wsems = (ws0, ws1)
        pending = {}
        for c in range(n_ch):
            b = c % 2
            if c >= 2:
                pending[(c - 2, 0)].wait()
                pending[(c - 2, 1)].wait()
            off = base + c * chunk
            pltpu.sync_copy(s0_hbm.at[pl.ds(off, chunk)], i0_v.at[b])
            pltpu.sync_copy(s1_hbm.at[pl.ds(off, chunk)], i1_v.at[b])
            pltpu.sync_copy(x_hbm.at[pl.ds(off, chunk)], rows_v.at[b])
            pending[(c, 0)] = pltpu.async_copy(
                rows_v.at[b], out_hbm.at[i0_v.at[b]], wsems[b])
            pending[(c, 1)] = pltpu.async_copy(
                rows_v.at[b], out_hbm.at[i1_v.at[b]], wsems[b])
        for c in range(max(0, n_ch - 2), n_ch):
            pending[(c, 0)].wait()
            pending[(c, 1)].wait()

    return dk(x, slot0, slot1)


def _sc_gather(table, idx, nrows, chunk):
    d = table.shape[1]
    b_per_w = nrows // _SC_WORKERS
    n_ch = b_per_w // chunk
    mesh = plsc.VectorSubcoreMesh(core_axis_name="c", subcore_axis_name="s")

    @functools.partial(
        pl.kernel,
        mesh=mesh,
        out_type=jax.ShapeDtypeStruct((nrows, d), jnp.float32),
        scratch_types=[
            pltpu.VMEM((2, chunk, d), jnp.float32),
            pltpu.VMEM((2, chunk), jnp.int32),
            pltpu.SemaphoreType.DMA,
            pltpu.SemaphoreType.DMA,
            pltpu.SemaphoreType.DMA,
        ],
    )
    def gk(table_hbm, idx_hbm, out_hbm, rows_v, idx_v, gs, ws0, ws1):
        wid = lax.axis_index("s") * 2 + lax.axis_index("c")
        base = wid * b_per_w
        wsems = (ws0, ws1)
        pending = {}
        for c in range(n_ch):
            b = c % 2
            if c >= 2:
                pending[c - 2].wait()
            off = base + c * chunk
            pltpu.sync_copy(idx_hbm.at[pl.ds(off, chunk)], idx_v.at[b])
            pltpu.async_copy(table_hbm.at[idx_v.at[b]], rows_v.at[b], gs).wait()
            pending[c] = pltpu.async_copy(
                rows_v.at[b], out_hbm.at[pl.ds(off, chunk)], wsems[b])
        for c in range(max(0, n_ch - 2), n_ch):
            pending[c].wait()

    return gk(table, idx)


def _moe_body(te_ref, lc_ref, x_ref, wgu_ref, wd_ref, out_ref):
    f = wd_ref.shape[2]
    hf = f // 2

    @pl.when(pl.program_id(0) < lc_ref[0])
    def _():
        x = x_ref[...]
        acc = None
        for j in range(2):
            wg = wgu_ref[0, j * hf:(j + 1) * hf, :]
            wu = wgu_ref[0, f + j * hf:f + (j + 1) * hf, :]
            gj = lax.dot_general(x, wg, (((1,), (1,)), ((), ())),
                                 preferred_element_type=jnp.float32)
            uj = lax.dot_general(x, wu, (((1,), (1,)), ((), ())),
                                 preferred_element_type=jnp.float32)
            hj = (gj * jax.nn.sigmoid(gj)) * uj
            wdj = wd_ref[0, :, j * hf:(j + 1) * hf]
            pj = lax.dot_general(hj, wdj, (((1,), (1,)), ((), ())),
                                 preferred_element_type=jnp.float32)
            acc = pj if acc is None else acc + pj
        out_ref[...] = acc


def _run_moe(tile_expert, live_tiles, x_sorted, w_gate_up, w_down, bt):
    np_, d = x_sorted.shape
    e, f2, _ = w_gate_up.shape
    f = f2 // 2
    nt = np_ // bt
    grid_spec = pltpu.PrefetchScalarGridSpec(
        num_scalar_prefetch=2,
        grid=(nt,),
        in_specs=[
            pl.BlockSpec((bt, d), lambda i, te, lc: (i, 0)),
            pl.BlockSpec((1, f2, d), lambda i, te, lc: (te[i], 0, 0)),
            pl.BlockSpec((1, d, f), lambda i, te, lc: (te[i], 0, 0)),
        ],
        out_specs=pl.BlockSpec((bt, d), lambda i, te, lc: (i, 0)),
    )
    return pl.pallas_call(
        _moe_body,
        grid_spec=grid_spec,
        out_shape=jax.ShapeDtypeStruct((np_, d), jnp.float32),
    )(tile_expert, live_tiles, x_sorted, w_gate_up, w_down)


def _shared_body(x_ref, wgu_ref, wd_ref, g_ref, out_ref):
    sf = wd_ref.shape[1]
    x = x_ref[...]
    wgu = wgu_ref[...]
    gu = lax.dot_general(x, wgu, (((1,), (1,)), ((), ())),
                         preferred_element_type=jnp.float32)
    g = gu[:, :sf]
    u = gu[:, sf:]
    h = (g * jax.nn.sigmoid(g)) * u
    wd = wd_ref[...]
    sh = lax.dot_general(h, wd, (((1,), (1,)), ((), ())),
                         preferred_element_type=jnp.float32)
    gate_logit = lax.dot_general(x, g_ref[...], (((1,), (1,)), ((), ())),
                                 preferred_element_type=jnp.float32)
    out_ref[...] = jax.nn.sigmoid(gate_logit) * sh


def _run_shared(x, sh_gate_up, sh_down, sh_gate, bs):
    t, d = x.shape
    sf2 = sh_gate_up.shape[0]
    sf = sh_down.shape[1]
    return pl.pallas_call(
        _shared_body,
        grid=(t // bs,),
        in_specs=[
            pl.BlockSpec((bs, d), lambda i: (i, 0)),
            pl.BlockSpec((sf2, d), lambda i: (0, 0)),
            pl.BlockSpec((d, sf), lambda i: (0, 0)),
            pl.BlockSpec((1, d), lambda i: (0, 0)),
        ],
        out_specs=pl.BlockSpec((bs, d), lambda i: (i, 0)),
        out_shape=jax.ShapeDtypeStruct((t, d), jnp.float32),
    )(x, sh_gate_up, sh_down, sh_gate)


def _combine_body(sh_ref, ye_ref, w_ref, out_ref):
    bs, d = sh_ref.shape
    ye = ye_ref[...]
    ye3 = ye.reshape(bs, 2, d)
    w = w_ref[...]
    out_ref[...] = (sh_ref[...]
                    + w[:, 0:1] * ye3[:, 0, :]
                    + w[:, 1:2] * ye3[:, 1, :])


def _run_combine(sh, ye_pairs, topk_w, bs):
    t, d = sh.shape
    return pl.pallas_call(
        _combine_body,
        grid=(t // bs,),
        in_specs=[
            pl.BlockSpec((bs, d), lambda i: (i, 0)),
            pl.BlockSpec((2 * bs, d), lambda i: (i, 0)),
            pl.BlockSpec((bs, 2), lambda i: (i, 0)),
        ],
        out_specs=pl.BlockSpec((bs, d), lambda i: (i, 0)),
        out_shape=jax.ShapeDtypeStruct((t, d), jnp.float32),
    )(sh, ye_pairs, topk_w)


def kernel(hidden_states, gate_w, w_gate_up, w_down, sh_gate_up, sh_down, sh_gate):
    b, s, d = hidden_states.shape
    e = gate_w.shape[0]
    t = b * s
    k = 2
    bt = 256
    nt = (t * k) // bt + e
    np_ = nt * bt

    x = hidden_states.reshape(t, d)

    topk_w, topk_ids = _run_router(x, gate_w, br=256)
    slot, tile_expert, live_tiles = _routing_metadata(topk_ids, e, bt, nt)
    slot2 = slot.reshape(t, k)
    slot0 = slot2[:, 0]
    slot1 = slot2[:, 1]

    x_sorted = _sc_dispatch(x, slot0, slot1, np_, chunk=16)
    sh = _run_shared(x, sh_gate_up, sh_down, sh_gate, bs=256)
    ye_sorted = _run_moe(tile_expert, live_tiles, x_sorted, w_gate_up, w_down, bt)
    ye_pairs = _sc_gather(ye_sorted, slot, t * k, chunk=16)

    out = _run_combine(sh, ye_pairs, topk_w, bs=256)
    return out.reshape(b, s, d)
